# Initial kernel scaffold; baseline (speedup 1.0000x reference)
import jax
import jax.numpy as jnp
from jax.experimental import pallas as pl


def kernel(x, position_weight, level_weight):
    raise NotImplementedError("write your pallas kernel here")


# speedup vs baseline: 30.4153x; 30.4153x over previous
import jax
import jax.numpy as jnp
from jax.experimental import pallas as pl
from jax.experimental.pallas import tpu as pltpu

_PATCH = 14
_NPOS = _PATCH * _PATCH
_NLEV = 256
_IB = 8


def _encoder_body(x_ref, pw_ref, lw_ref, out_ref):
    pw = pw_ref[...]
    lw = lw_ref[...]
    iota = jax.lax.broadcasted_iota(jnp.int32, (_NPOS, _NLEV), 1)
    for i in range(_IB):
        x_i = x_ref[i]
        idx = jnp.clip(jnp.round(x_i * (_NLEV - 1)), 0.0,
                       float(_NLEV - 1)).astype(jnp.int32)
        counts = jnp.zeros((_NPOS, _NLEV), jnp.bfloat16)
        for p in range(4):
            counts += (idx[p][:, None] == iota).astype(jnp.bfloat16)
        m = jax.lax.dot_general(
            counts, lw, (((1,), (0,)), ((), ())),
            preferred_element_type=jnp.float32,
        )
        s = jnp.sum(m * pw, axis=0)
        out_ref[i, :] = jnp.where(s > 0.0, 1.0, -1.0)


def kernel(x, position_weight, level_weight):
    B, C, H, W = x.shape
    p = _PATCH
    D = position_weight.shape[1]
    x_pj = x.reshape(B, C, H // p, p, W // p, p)
    x_pj = x_pj.transpose(0, 1, 2, 4, 3, 5).reshape(B, 4, p * p)
    lw_bf16 = level_weight.astype(jnp.bfloat16)

    grid = (B // _IB,)
    return pl.pallas_call(
        _encoder_body,
        grid=grid,
        in_specs=[
            pl.BlockSpec((_IB, 4, _NPOS), lambda i: (i, 0, 0)),
            pl.BlockSpec((_NPOS, D), lambda i: (0, 0)),
            pl.BlockSpec((_NLEV, D), lambda i: (0, 0)),
        ],
        out_specs=pl.BlockSpec((_IB, D), lambda i: (i, 0)),
        out_shape=jax.ShapeDtypeStruct((B, D), jnp.float32),
    )(x_pj, position_weight, lw_bf16)
